# baseline (speedup 1.0000x reference)
import functools

import jax
import jax.numpy as jnp
from jax import lax
from jax.experimental import pallas as pl
from jax.experimental.pallas import tpu as pltpu
from jax.experimental.pallas import tpu_sc as plsc

N_NODES = 10000
N_PAD = 10240
N_EDGES = 320000
NW = 32
CHUNK = 128
ITERS = 80
E_PAD = NW * CHUNK * ITERS
ROWS_PER_TILE = N_PAD // 16
K = 64
B_GRAPHS = 100
HIDDEN = 128
TOTAL_LATENT = 385


HALF = ITERS // 2
BOUNCE = CHUNK
NH0, NH1 = 1, 3


def _make_sc_scatter(gather_rows):
    mesh = plsc.VectorSubcoreMesh(core_axis_name="c", subcore_axis_name="s",
                                  num_cores=2, num_subcores=16)
    out_type = jax.ShapeDtypeStruct((2 * N_PAD, HIDDEN), jnp.float32)
    scratch = [
        pltpu.VMEM((HALF, CHUNK), jnp.int32),
        pltpu.VMEM((HALF, CHUNK), jnp.int32),
        pltpu.VMEM((CHUNK, HIDDEN), jnp.float32),
        pltpu.VMEM((CHUNK, HIDDEN), jnp.float32),
        pltpu.VMEM_SHARED((N_PAD, HIDDEN), jnp.float32),
        pltpu.SemaphoreType.DMA,
        pltpu.SemaphoreType.DMA,
        pltpu.SemaphoreType.DMA,
    ]

    def body(h_hbm, src_hbm, dst_hbm, z_hbm, out_hbm,
             idxd_v, idxs_v, buf_a, buf_b, acc_sh, sem_a, sem_b, sem_s):
        c = lax.axis_index("c")
        s = lax.axis_index("s")
        row0 = s * ROWS_PER_TILE
        def zinit(j, _):
            r = j * BOUNCE
            pltpu.sync_copy(z_hbm.at[pl.ds(r, BOUNCE)], buf_a)
            pltpu.sync_copy(buf_a, acc_sh.at[pl.ds(row0 + r, BOUNCE)])
            return 0
        lax.fori_loop(0, ROWS_PER_TILE // BOUNCE, zinit, 0)
        plsc.subcore_barrier()

        wid = s * 2 + c

        def scat_block(b, buf):
            pltpu.async_copy(buf, acc_sh.at[idxd_v.at[b]], sem_s,
                             add=True).wait()

        if gather_rows:
            def gath_block(b, buf, sem):
                pltpu.async_copy(h_hbm.at[idxs_v.at[b]], buf, sem)

            def drain(buf, sem):
                pltpu.make_async_copy(h_hbm.at[idxs_v.at[0]], buf, sem).wait()

            nh = jnp.where(c == 0, NH0, NH1)

            def outer(half, _):
                irow0 = s * (2 * ITERS) + c * (NH0 * HALF) + half * HALF
                pltpu.sync_copy(dst_hbm.at[pl.ds(irow0, HALF)], idxd_v)
                pltpu.sync_copy(src_hbm.at[pl.ds(irow0, HALF)], idxs_v)
                gath_block(0, buf_a, sem_a)

                def pair(i, _):
                    b0 = 2 * i
                    drain(buf_a, sem_a)
                    gath_block(b0 + 1, buf_b, sem_b)
                    scat_block(b0, buf_a)
                    drain(buf_b, sem_b)

                    @pl.when(i < HALF // 2 - 1)
                    def _():
                        gath_block(b0 + 2, buf_a, sem_a)
                    scat_block(b0 + 1, buf_b)
                    return 0
                lax.fori_loop(0, HALF // 2, pair, 0)
                return 0
            lax.fori_loop(0, nh, outer, 0)
        else:
            pltpu.sync_copy(h_hbm.at[pl.ds(0, CHUNK)], buf_a)
            for half in range(2):
                irow0 = wid * ITERS + half * HALF
                pltpu.sync_copy(dst_hbm.at[pl.ds(irow0, HALF)], idxd_v)

                def blk(b, _):
                    scat_block(b, buf_a)
                    return 0
                lax.fori_loop(0, HALF, blk, 0)

        plsc.subcore_barrier()
        def wback(j, _):
            r = j * BOUNCE
            pltpu.sync_copy(acc_sh.at[pl.ds(row0 + r, BOUNCE)], buf_a)
            pltpu.sync_copy(buf_a,
                            out_hbm.at[pl.ds(c * N_PAD + row0 + r, BOUNCE)])
            return 0
        lax.fori_loop(0, ROWS_PER_TILE // BOUNCE, wback, 0)

    return pl.kernel(body, out_type=out_type, mesh=mesh, scratch_types=scratch)


BLK = 1024
GRID = N_PAD // BLK


def _mm_body(x_ref, w_ref, b_ref, o_ref):
    o_ref[...] = lax.dot_general(
        x_ref[...], w_ref[...], (((1,), (1,)), ((), ())),
        preferred_element_type=jnp.float32) + b_ref[...]


def _matmul(x, w, b):
    return pl.pallas_call(
        _mm_body,
        grid=(GRID,),
        in_specs=[
            pl.BlockSpec((BLK, x.shape[1]), lambda i: (i, 0)),
            pl.BlockSpec(w.shape, lambda i: (0, 0)),
            pl.BlockSpec(b.shape, lambda i: (0, 0)),
        ],
        out_specs=pl.BlockSpec((BLK, w.shape[0]), lambda i: (i, 0)),
        out_shape=jax.ShapeDtypeStruct((N_PAD, w.shape[0]), jnp.float32),
    )(x, w, b)


def _fuse_mm_body(h_ref, sa_ref, sb_ref, da_ref, db_ref, w_ref, b_ref, o_ref):
    deg = 1.0 + da_ref[...][:, 0:1] + db_ref[...][:, 0:1]
    xv = jnp.tanh((h_ref[...] + sa_ref[...] + sb_ref[...]) / deg)
    o_ref[...] = lax.dot_general(
        xv, w_ref[...], (((1,), (1,)), ((), ())),
        preferred_element_type=jnp.float32) + b_ref[...]


def _fused_layer(h, sa, sb, da, db, w, b, out_width):
    return pl.pallas_call(
        _fuse_mm_body,
        grid=(GRID,),
        in_specs=[
            pl.BlockSpec((BLK, HIDDEN), lambda i: (i, 0)),
            pl.BlockSpec((BLK, HIDDEN), lambda i: (i, 0)),
            pl.BlockSpec((BLK, HIDDEN), lambda i: (i, 0)),
            pl.BlockSpec((BLK, 16), lambda i: (i, 0)),
            pl.BlockSpec((BLK, 16), lambda i: (i, 0)),
            pl.BlockSpec(w.shape, lambda i: (0, 0)),
            pl.BlockSpec(b.shape, lambda i: (0, 0)),
        ],
        out_specs=pl.BlockSpec((BLK, out_width), lambda i: (i, 0)),
        out_shape=jax.ShapeDtypeStruct((N_PAD, out_width), jnp.float32),
    )(h, sa, sb, da, db, w, b)


def _topk_body(batch_ref, h3_ref, s3a_ref, s3b_ref, da_ref, db_ref, o_ref):
    deg = 1.0 + da_ref[...] + db_ref[...]
    v = jnp.tanh((h3_ref[...] + s3a_ref[...] + s3b_ref[...]) / deg)
    n = v.shape[1]
    gid = lax.broadcasted_iota(jnp.int32, (B_GRAPHS, n), 0)
    bvec = jnp.broadcast_to(batch_ref[...], (B_GRAPHS, n))
    neg = jnp.float32(-jnp.inf)
    dense = jnp.where(bvec == gid, jnp.broadcast_to(v, (B_GRAPHS, n)), neg)
    nid = lax.broadcasted_iota(jnp.int32, (B_GRAPHS, n), 1)
    big = jnp.int32(n)
    kiota = lax.broadcasted_iota(jnp.int32, (B_GRAPHS, K), 1)

    def round_(r, carry):
        d, out = carry
        m = jnp.max(d, axis=1, keepdims=True)
        eq = d == m
        first = jnp.min(jnp.where(eq, nid, big), axis=1, keepdims=True)
        mval = jnp.where(m == neg, 0.0, m)
        out = jnp.where(kiota == r, jnp.broadcast_to(mval, (B_GRAPHS, K)), out)
        return jnp.where(nid == first, neg, d), out

    _, res = lax.fori_loop(0, K, round_, (dense, jnp.zeros((B_GRAPHS, K), jnp.float32)))
    o_ref[...] = res


def _topk(batch2d, h3c, s3a, s3b, da, db):
    return pl.pallas_call(
        _topk_body,
        out_shape=jax.ShapeDtypeStruct((B_GRAPHS, K), jnp.float32),
    )(batch2d, h3c, s3a, s3b, da, db)


def _tail_body(vt_ref, cw1_ref, cb1_ref, cw2_ref, cb2_ref,
               f1w_ref, f1b_ref, f2w_ref, f2b_ref, o_ref):
    vt = vt_ref[...]
    cw1 = cw1_ref[...]
    cb1 = cb1_ref[...]
    a1 = lax.dot_general(vt, cw1[:, :K], (((1,), (1,)), ((), ())),
                         preferred_element_type=jnp.float32) + cb1
    a1 = jnp.maximum(a1, 0.0)
    c1 = jnp.maximum(cb1, 0.0)
    m0 = jnp.maximum(a1, c1)
    cw2 = cw2_ref[...]
    cb2 = cb2_ref[...]
    w20 = cw2[:, :, 0]
    wsum = jnp.sum(cw2, axis=2)
    wrest = wsum - w20
    t_rest = lax.dot_general(c1, wrest, (((1,), (1,)), ((), ())),
                             preferred_element_type=jnp.float32)
    out0 = lax.dot_general(m0, w20, (((1,), (1,)), ((), ())),
                           preferred_element_type=jnp.float32) + t_rest + cb2
    out0 = jnp.maximum(out0, 0.0)
    cv = lax.dot_general(c1, wsum, (((1,), (1,)), ((), ())),
                         preferred_element_type=jnp.float32) + cb2
    cv = jnp.maximum(cv, 0.0)
    f1w = f1w_ref[...]
    g0 = f1w[:, :, 0]
    grest = jnp.sum(f1w, axis=2) - g0
    cc = lax.dot_general(cv, grest, (((1,), (1,)), ((), ())),
                         preferred_element_type=jnp.float32)
    y1 = lax.dot_general(out0, g0, (((1,), (1,)), ((), ())),
                         preferred_element_type=jnp.float32) + cc + f1b_ref[...]
    y1 = jnp.maximum(y1, 0.0)
    o_ref[...] = lax.dot_general(
        y1, f2w_ref[...], (((1,), (1,)), ((), ())),
        preferred_element_type=jnp.float32) + f2b_ref[...]


def _tail(vt, cw1, cb1, cw2, cb2, f1w, f1b, f2w, f2b):
    return pl.pallas_call(
        _tail_body,
        out_shape=jax.ShapeDtypeStruct((B_GRAPHS, f2w.shape[0]), jnp.float32),
    )(vt, cw1, cb1, cw2, cb2, f1w, f1b, f2w, f2b)


@functools.lru_cache(maxsize=None)
def _get_sc_scatter(gather_rows):
    return _make_sc_scatter(gather_rows)


def kernel(x, edge_index, batch, W1, b1, W2, b2, W3, b3,
           cw1, cb1, cw2, cb2, f1w, f1b, f2w, f2b):
    f32 = jnp.float32
    pad_e = E_PAD - N_EDGES
    src = jnp.concatenate([edge_index[0], jnp.full((pad_e,), N_NODES, jnp.int32)])
    dst = jnp.concatenate([edge_index[1], jnp.full((pad_e,), N_NODES, jnp.int32)])
    src = src.reshape(-1, CHUNK)
    dst = dst.reshape(-1, CHUNK)
    xp = jnp.concatenate([x, jnp.zeros((N_PAD - N_NODES, x.shape[1]), f32)])
    z = jnp.zeros((ROWS_PER_TILE, HIDDEN), f32)
    ones = jnp.ones((CHUNK, HIDDEN), f32)

    degp = _get_sc_scatter(False)(ones, src, src, z)
    da, db = degp[:N_PAD, :16], degp[N_PAD:, :16]

    h1 = _matmul(xp, W1, b1.reshape(1, -1))
    s1 = _get_sc_scatter(True)(h1, src, dst, z)
    s1a, s1b = s1[:N_PAD], s1[N_PAD:]

    h2 = _fused_layer(h1, s1a, s1b, da, db, W2, b2.reshape(1, -1), HIDDEN)
    s2 = _get_sc_scatter(True)(h2, src, dst, z)
    s2a, s2b = s2[:N_PAD], s2[N_PAD:]

    w3r = jnp.broadcast_to(W3, (HIDDEN, HIDDEN))
    b3r = jnp.broadcast_to(b3.reshape(1, 1), (1, HIDDEN))
    h3 = _fused_layer(h2, s2a, s2b, da, db, w3r, b3r, HIDDEN)
    s3 = _get_sc_scatter(True)(h3, src, dst, z)

    def row(a):
        return a[:N_NODES, 0:1].reshape(1, N_NODES)
    batch2d = batch.reshape(1, N_NODES)
    vt = _topk(batch2d, row(h3), row(s3[:N_PAD]), row(s3[N_PAD:]),
               row(da), row(db))

    return _tail(vt, cw1[:, 0, :], cb1.reshape(1, -1),
                 cw2, cb2.reshape(1, -1),
                 f1w.reshape(128, 32, 28), f1b.reshape(1, -1),
                 f2w, f2b.reshape(1, -1))


# speedup vs baseline: 1.2848x; 1.2848x over previous
import functools

import jax
import jax.numpy as jnp
from jax import lax
from jax.experimental import pallas as pl
from jax.experimental.pallas import tpu as pltpu
from jax.experimental.pallas import tpu_sc as plsc

N_NODES = 10000
N_PAD = 10240
N_EDGES = 320000
NW = 32
CHUNK = 128
ITERS = 80
E_PAD = NW * CHUNK * ITERS
ROWS_PER_TILE = N_PAD // 16
K = 64
B_GRAPHS = 100
HIDDEN = 128
TOTAL_LATENT = 385


HALF = ITERS // 2
BOUNCE = CHUNK
NH0, NH1 = 3, 1


def _make_sc_scatter(gather_rows):
    mesh = plsc.VectorSubcoreMesh(core_axis_name="c", subcore_axis_name="s",
                                  num_cores=2, num_subcores=16)
    out_type = jax.ShapeDtypeStruct((2 * N_PAD, HIDDEN), jnp.float32)
    scratch = [
        pltpu.VMEM((HALF, CHUNK), jnp.int32),
        pltpu.VMEM((HALF, CHUNK), jnp.int32),
        pltpu.VMEM((CHUNK, HIDDEN), jnp.float32),
        pltpu.VMEM((CHUNK, HIDDEN), jnp.float32),
        pltpu.VMEM_SHARED((N_PAD, HIDDEN), jnp.float32),
        pltpu.SemaphoreType.DMA,
        pltpu.SemaphoreType.DMA,
        pltpu.SemaphoreType.DMA,
    ]

    def body(h_hbm, src_hbm, dst_hbm, z_hbm, out_hbm,
             idxd_v, idxs_v, buf_a, buf_b, acc_sh, sem_a, sem_b, sem_s):
        c = lax.axis_index("c")
        s = lax.axis_index("s")
        row0 = s * ROWS_PER_TILE
        def zinit(j, _):
            r = j * BOUNCE
            pltpu.sync_copy(z_hbm.at[pl.ds(r, BOUNCE)], buf_a)
            pltpu.sync_copy(buf_a, acc_sh.at[pl.ds(row0 + r, BOUNCE)])
            return 0
        lax.fori_loop(0, ROWS_PER_TILE // BOUNCE, zinit, 0)
        plsc.subcore_barrier()

        wid = s * 2 + c

        def scat_block(b, buf):
            pltpu.async_copy(buf, acc_sh.at[idxd_v.at[b]], sem_s,
                             add=True).wait()

        if gather_rows:
            def gath_block(b, buf, sem):
                pltpu.async_copy(h_hbm.at[idxs_v.at[b]], buf, sem)

            def drain(buf, sem):
                pltpu.make_async_copy(h_hbm.at[idxs_v.at[0]], buf, sem).wait()

            nh = jnp.where(c == 0, NH0, NH1)

            def outer(half, _):
                irow0 = s * (2 * ITERS) + c * (NH0 * HALF) + half * HALF
                pltpu.sync_copy(dst_hbm.at[pl.ds(irow0, HALF)], idxd_v)
                pltpu.sync_copy(src_hbm.at[pl.ds(irow0, HALF)], idxs_v)
                gath_block(0, buf_a, sem_a)

                def pair(i, _):
                    b0 = 2 * i
                    drain(buf_a, sem_a)
                    gath_block(b0 + 1, buf_b, sem_b)
                    scat_block(b0, buf_a)
                    drain(buf_b, sem_b)

                    @pl.when(i < HALF // 2 - 1)
                    def _():
                        gath_block(b0 + 2, buf_a, sem_a)
                    scat_block(b0 + 1, buf_b)
                    return 0
                lax.fori_loop(0, HALF // 2, pair, 0)
                return 0
            lax.fori_loop(0, nh, outer, 0)
        else:
            pltpu.sync_copy(h_hbm.at[pl.ds(0, CHUNK)], buf_a)
            for half in range(2):
                irow0 = wid * ITERS + half * HALF
                pltpu.sync_copy(dst_hbm.at[pl.ds(irow0, HALF)], idxd_v)

                def blk(b, _):
                    scat_block(b, buf_a)
                    return 0
                lax.fori_loop(0, HALF, blk, 0)

        plsc.subcore_barrier()
        def wback(j, _):
            r = j * BOUNCE
            pltpu.sync_copy(acc_sh.at[pl.ds(row0 + r, BOUNCE)], buf_a)
            pltpu.sync_copy(buf_a,
                            out_hbm.at[pl.ds(c * N_PAD + row0 + r, BOUNCE)])
            return 0
        lax.fori_loop(0, ROWS_PER_TILE // BOUNCE, wback, 0)

    return pl.kernel(body, out_type=out_type, mesh=mesh, scratch_types=scratch)


BLK = 1024
GRID = N_PAD // BLK


def _mm_body(x_ref, w_ref, b_ref, o_ref):
    o_ref[...] = lax.dot_general(
        x_ref[...], w_ref[...], (((1,), (1,)), ((), ())),
        preferred_element_type=jnp.float32) + b_ref[...]


def _matmul(x, w, b):
    return pl.pallas_call(
        _mm_body,
        grid=(GRID,),
        in_specs=[
            pl.BlockSpec((BLK, x.shape[1]), lambda i: (i, 0)),
            pl.BlockSpec(w.shape, lambda i: (0, 0)),
            pl.BlockSpec(b.shape, lambda i: (0, 0)),
        ],
        out_specs=pl.BlockSpec((BLK, w.shape[0]), lambda i: (i, 0)),
        out_shape=jax.ShapeDtypeStruct((N_PAD, w.shape[0]), jnp.float32),
    )(x, w, b)


def _fuse_mm_body(h_ref, sa_ref, sb_ref, da_ref, db_ref, w_ref, b_ref, o_ref):
    deg = 1.0 + da_ref[...][:, 0:1] + db_ref[...][:, 0:1]
    xv = jnp.tanh((h_ref[...] + sa_ref[...] + sb_ref[...]) / deg)
    o_ref[...] = lax.dot_general(
        xv, w_ref[...], (((1,), (1,)), ((), ())),
        preferred_element_type=jnp.float32) + b_ref[...]


def _fused_layer(h, sa, sb, da, db, w, b, out_width):
    return pl.pallas_call(
        _fuse_mm_body,
        grid=(GRID,),
        in_specs=[
            pl.BlockSpec((BLK, HIDDEN), lambda i: (i, 0)),
            pl.BlockSpec((BLK, HIDDEN), lambda i: (i, 0)),
            pl.BlockSpec((BLK, HIDDEN), lambda i: (i, 0)),
            pl.BlockSpec((BLK, 16), lambda i: (i, 0)),
            pl.BlockSpec((BLK, 16), lambda i: (i, 0)),
            pl.BlockSpec(w.shape, lambda i: (0, 0)),
            pl.BlockSpec(b.shape, lambda i: (0, 0)),
        ],
        out_specs=pl.BlockSpec((BLK, out_width), lambda i: (i, 0)),
        out_shape=jax.ShapeDtypeStruct((N_PAD, out_width), jnp.float32),
    )(h, sa, sb, da, db, w, b)


def _topk_body(batch_ref, h3_ref, s3a_ref, s3b_ref, da_ref, db_ref, o_ref):
    deg = 1.0 + da_ref[...] + db_ref[...]
    v = jnp.tanh((h3_ref[...] + s3a_ref[...] + s3b_ref[...]) / deg)
    n = v.shape[1]
    gid = lax.broadcasted_iota(jnp.int32, (B_GRAPHS, n), 0)
    bvec = jnp.broadcast_to(batch_ref[...], (B_GRAPHS, n))
    neg = jnp.float32(-jnp.inf)
    dense = jnp.where(bvec == gid, jnp.broadcast_to(v, (B_GRAPHS, n)), neg)
    nid = lax.broadcasted_iota(jnp.int32, (B_GRAPHS, n), 1)
    big = jnp.int32(n)
    kiota = lax.broadcasted_iota(jnp.int32, (B_GRAPHS, K), 1)

    def round_(r, carry):
        d, out = carry
        m = jnp.max(d, axis=1, keepdims=True)
        eq = d == m
        first = jnp.min(jnp.where(eq, nid, big), axis=1, keepdims=True)
        mval = jnp.where(m == neg, 0.0, m)
        out = jnp.where(kiota == r, jnp.broadcast_to(mval, (B_GRAPHS, K)), out)
        return jnp.where(nid == first, neg, d), out

    _, res = lax.fori_loop(0, K, round_, (dense, jnp.zeros((B_GRAPHS, K), jnp.float32)))
    o_ref[...] = res


def _topk(batch2d, h3c, s3a, s3b, da, db):
    return pl.pallas_call(
        _topk_body,
        out_shape=jax.ShapeDtypeStruct((B_GRAPHS, K), jnp.float32),
    )(batch2d, h3c, s3a, s3b, da, db)


def _tail_body(vt_ref, cw1_ref, cb1_ref, cw2_ref, cb2_ref,
               f1w_ref, f1b_ref, f2w_ref, f2b_ref, o_ref):
    vt = vt_ref[...]
    cw1 = cw1_ref[...]
    cb1 = cb1_ref[...]
    a1 = lax.dot_general(vt, cw1[:, :K], (((1,), (1,)), ((), ())),
                         preferred_element_type=jnp.float32) + cb1
    a1 = jnp.maximum(a1, 0.0)
    c1 = jnp.maximum(cb1, 0.0)
    m0 = jnp.maximum(a1, c1)
    cw2 = cw2_ref[...]
    cb2 = cb2_ref[...]
    w20 = cw2[:, :, 0]
    wsum = jnp.sum(cw2, axis=2)
    wrest = wsum - w20
    t_rest = lax.dot_general(c1, wrest, (((1,), (1,)), ((), ())),
                             preferred_element_type=jnp.float32)
    out0 = lax.dot_general(m0, w20, (((1,), (1,)), ((), ())),
                           preferred_element_type=jnp.float32) + t_rest + cb2
    out0 = jnp.maximum(out0, 0.0)
    cv = lax.dot_general(c1, wsum, (((1,), (1,)), ((), ())),
                         preferred_element_type=jnp.float32) + cb2
    cv = jnp.maximum(cv, 0.0)
    f1w = f1w_ref[...]
    g0 = f1w[:, :, 0]
    grest = jnp.sum(f1w, axis=2) - g0
    cc = lax.dot_general(cv, grest, (((1,), (1,)), ((), ())),
                         preferred_element_type=jnp.float32)
    y1 = lax.dot_general(out0, g0, (((1,), (1,)), ((), ())),
                         preferred_element_type=jnp.float32) + cc + f1b_ref[...]
    y1 = jnp.maximum(y1, 0.0)
    o_ref[...] = lax.dot_general(
        y1, f2w_ref[...], (((1,), (1,)), ((), ())),
        preferred_element_type=jnp.float32) + f2b_ref[...]


def _tail(vt, cw1, cb1, cw2, cb2, f1w, f1b, f2w, f2b):
    return pl.pallas_call(
        _tail_body,
        out_shape=jax.ShapeDtypeStruct((B_GRAPHS, f2w.shape[0]), jnp.float32),
    )(vt, cw1, cb1, cw2, cb2, f1w, f1b, f2w, f2b)


@functools.lru_cache(maxsize=None)
def _get_sc_scatter(gather_rows):
    return _make_sc_scatter(gather_rows)


def kernel(x, edge_index, batch, W1, b1, W2, b2, W3, b3,
           cw1, cb1, cw2, cb2, f1w, f1b, f2w, f2b):
    f32 = jnp.float32
    pad_e = E_PAD - N_EDGES
    src = jnp.concatenate([edge_index[0], jnp.full((pad_e,), N_NODES, jnp.int32)])
    dst = jnp.concatenate([edge_index[1], jnp.full((pad_e,), N_NODES, jnp.int32)])
    src = src.reshape(-1, CHUNK)
    dst = dst.reshape(-1, CHUNK)
    xp = jnp.concatenate([x, jnp.zeros((N_PAD - N_NODES, x.shape[1]), f32)])
    z = jnp.zeros((ROWS_PER_TILE, HIDDEN), f32)
    ones = jnp.ones((CHUNK, HIDDEN), f32)

    degp = _get_sc_scatter(False)(ones, src, src, z)
    da, db = degp[:N_PAD, :16], degp[N_PAD:, :16]

    h1 = _matmul(xp, W1, b1.reshape(1, -1))
    s1 = _get_sc_scatter(True)(h1, src, dst, z)
    s1a, s1b = s1[:N_PAD], s1[N_PAD:]

    h2 = _fused_layer(h1, s1a, s1b, da, db, W2, b2.reshape(1, -1), HIDDEN)
    s2 = _get_sc_scatter(True)(h2, src, dst, z)
    s2a, s2b = s2[:N_PAD], s2[N_PAD:]

    w3r = jnp.broadcast_to(W3, (HIDDEN, HIDDEN))
    b3r = jnp.broadcast_to(b3.reshape(1, 1), (1, HIDDEN))
    h3 = _fused_layer(h2, s2a, s2b, da, db, w3r, b3r, HIDDEN)
    s3 = _get_sc_scatter(True)(h3, src, dst, z)

    def row(a):
        return a[:N_NODES, 0:1].reshape(1, N_NODES)
    batch2d = batch.reshape(1, N_NODES)
    vt = _topk(batch2d, row(h3), row(s3[:N_PAD]), row(s3[N_PAD:]),
               row(da), row(db))

    return _tail(vt, cw1[:, 0, :], cb1.reshape(1, -1),
                 cw2, cb2.reshape(1, -1),
                 f1w.reshape(128, 32, 28), f1b.reshape(1, -1),
                 f2w, f2b.reshape(1, -1))
